# baseline (speedup 1.0000x reference)
import functools

import jax
import jax.numpy as jnp
from jax import lax
from jax.experimental import pallas as pl
from jax.experimental.pallas import tpu as pltpu
from jax.experimental.pallas import tpu_sc as plsc

_ROWS = 65536
_BATCH = 256
_BR = 1024
_NB = _ROWS // _BR
_NCORE = 2
_PER_C = _NB // _NCORE
_NBUF = 6
_LOOK = 3


def _body(x_ref, o_ref, *rest):
    bufs = rest[:_NBUF]
    lsems = rest[_NBUF:2 * _NBUF]
    ssems = rest[2 * _NBUF:3 * _NBUF]
    cid = lax.axis_index("c")
    base = cid * _PER_C

    def load(i, b):
        src = pl.multiple_of(((base + i) ^ 1) * _BR, _BR)
        return pltpu.make_async_copy(
            x_ref.at[pl.ds(src, _BR)], bufs[b], lsems[b])

    def store(i, b):
        dst = pl.multiple_of((base + i) * _BR, _BR)
        return pltpu.make_async_copy(
            bufs[b], o_ref.at[pl.ds(dst, _BR)], ssems[b])

    for i in range(_LOOK):
        load(i, i % _NBUF).start()
    for i in range(_PER_C):
        b = i % _NBUF
        nxt = i + _LOOK
        if nxt < _PER_C:
            bn = nxt % _NBUF
            if nxt >= _NBUF:
                store(nxt - _NBUF, bn).wait()
            load(nxt, bn).start()
        load(i, b).wait()
        store(i, b).start()
    for i in range(_PER_C - _NBUF, _PER_C):
        store(i, i % _NBUF).wait()


@functools.partial(jax.jit, donate_argnums=())
def _scs_swap(x):
    mesh = plsc.ScalarSubcoreMesh(axis_name="c", num_cores=_NCORE)
    scratch = [pltpu.VMEM_SHARED((_BR, _BATCH), jnp.float32)
               for _ in range(_NBUF)]
    scratch += [pltpu.SemaphoreType.DMA for _ in range(2 * _NBUF)]
    return pl.kernel(
        _body,
        mesh=mesh,
        out_type=jax.ShapeDtypeStruct((_ROWS, _BATCH), jnp.float32),
        scratch_types=scratch,
    )(x)


def kernel(x):
    return _scs_swap(x)


# speedup vs baseline: 1.4657x; 1.4657x over previous
import functools

import jax
import jax.numpy as jnp
from jax.experimental import pallas as pl
from jax.experimental.pallas import tpu as pltpu

_ROWS = 65536
_BATCH = 256
_BR = 1024
_NB = _ROWS // _BR
_NBUF = 32
_LOOK = 16


def _body(x_ref, o_ref, *rest):
    bufs = rest[:_NBUF]
    lsems = rest[_NBUF:2 * _NBUF]
    ssems = rest[2 * _NBUF:3 * _NBUF]

    def load(i, b):
        return pltpu.make_async_copy(
            x_ref.at[pl.ds((i ^ 1) * _BR, _BR)], bufs[b], lsems[b])

    def store(i, b):
        return pltpu.make_async_copy(
            bufs[b], o_ref.at[pl.ds(i * _BR, _BR)], ssems[b])

    for i in range(_LOOK):
        load(i, i % _NBUF).start()
    for i in range(_NB):
        b = i % _NBUF
        nxt = i + _LOOK
        if nxt < _NB:
            bn = nxt % _NBUF
            if nxt >= _NBUF:
                store(nxt - _NBUF, bn).wait()
            load(nxt, bn).start()
        load(i, b).wait()
        store(i, b).start()
    for i in range(_NB - _NBUF, _NB):
        store(i, i % _NBUF).wait()


@functools.partial(jax.jit, donate_argnums=())
def _dma_ring_swap(x):
    scratch = [pltpu.VMEM((_BR, _BATCH), jnp.float32) for _ in range(_NBUF)]
    scratch += [pltpu.SemaphoreType.DMA for _ in range(2 * _NBUF)]
    return pl.pallas_call(
        _body,
        in_specs=[pl.BlockSpec(memory_space=pl.ANY)],
        out_specs=pl.BlockSpec(memory_space=pl.ANY),
        out_shape=jax.ShapeDtypeStruct((_ROWS, _BATCH), jnp.float32),
        scratch_shapes=scratch,
    )(x)


def kernel(x):
    return _dma_ring_swap(x)


# speedup vs baseline: 1.4793x; 1.0093x over previous
import functools

import jax
import jax.numpy as jnp
from jax.experimental import pallas as pl
from jax.experimental.pallas import tpu as pltpu

_ROWS = 65536
_BATCH = 256
_BR = 1024
_NB = _ROWS // _BR
_NBUF = 12
_LOOK = 6


def _body(x_ref, o_ref, *rest):
    bufs = rest[:_NBUF]
    lsems = rest[_NBUF:2 * _NBUF]
    ssems = rest[2 * _NBUF:3 * _NBUF]

    def load(i, b):
        return pltpu.make_async_copy(
            x_ref.at[pl.ds((i ^ 1) * _BR, _BR)], bufs[b], lsems[b])

    def store(i, b):
        return pltpu.make_async_copy(
            bufs[b], o_ref.at[pl.ds(i * _BR, _BR)], ssems[b])

    for i in range(_LOOK):
        load(i, i % _NBUF).start()
    for i in range(_NB):
        b = i % _NBUF
        nxt = i + _LOOK
        if nxt < _NB:
            bn = nxt % _NBUF
            if nxt >= _NBUF:
                store(nxt - _NBUF, bn).wait()
            load(nxt, bn).start()
        load(i, b).wait()
        store(i, b).start()
    for i in range(_NB - _NBUF, _NB):
        store(i, i % _NBUF).wait()


@functools.partial(jax.jit, donate_argnums=())
def _dma_ring_swap(x):
    scratch = [pltpu.VMEM((_BR, _BATCH), jnp.float32) for _ in range(_NBUF)]
    scratch += [pltpu.SemaphoreType.DMA for _ in range(2 * _NBUF)]
    return pl.pallas_call(
        _body,
        in_specs=[pl.BlockSpec(memory_space=pl.ANY)],
        out_specs=pl.BlockSpec(memory_space=pl.ANY),
        out_shape=jax.ShapeDtypeStruct((_ROWS, _BATCH), jnp.float32),
        scratch_shapes=scratch,
    )(x)


def kernel(x):
    return _dma_ring_swap(x)
